# baseline (speedup 1.0000x reference)
import jax
import jax.numpy as jnp
from jax import lax
from jax.experimental import pallas as pl
from jax.experimental.pallas import tpu as pltpu
from jax.experimental.pallas import tpu_sc as plsc

_ROWS, _COLS = 1024, 128
_L = 16
_NC, _NS = 2, 16
_NW = _NC * _NS
_RPW = 16
_SC_BLOCKS = _NW // 2
_SC_ROWS = _SC_BLOCKS * _RPW
_WVALS = _RPW * _COLS
_JU, _KU = 8, 4
_JHALF = _COLS // 2
_TC_BLK = 64


def _sc_pair_loss(yp_hbm, yt_hbm, loss_hbm, pairs_hbm,
                  ypv, ytv, av, nbv, lossv, pairsv):
    wid = lax.axis_index("s") * _NC + lax.axis_index("c")
    blk = wid // 2
    half = wid % 2
    jlo = half * _JHALF
    pltpu.sync_copy(yp_hbm.at[pl.ds(blk * _WVALS, _WVALS)], ypv)
    pltpu.sync_copy(yt_hbm.at[pl.ds(blk * _WVALS, _WVALS)], ytv)

    zeros = jnp.zeros((_L,), jnp.float32)
    row_off = lax.iota(jnp.int32, _L) * _COLS

    def prep(j, carry):
        pc, nc = carry
        idx = row_off + j
        y = plsc.load_gather(ypv, [idx])
        t = plsc.load_gather(ytv, [idx])
        is_pos = t == 1
        is_neg = t == 0
        av[pl.ds(j * _L, _L)] = jnp.where(is_pos, y, jnp.inf)
        nbv[pl.ds(j * _L, _L)] = jnp.where(is_neg, y + 1.0, -jnp.inf)
        in_half = ((j >= jlo) & (j < jlo + _JHALF)).astype(jnp.float32)
        pc = pc + jnp.where(is_pos, 1.0, 0.0) * in_half
        nc = nc + jnp.where(is_neg, 1.0, 0.0)
        return pc, nc

    pos_cnt, neg_cnt = lax.fori_loop(0, _COLS, prep, (zeros, zeros))

    def jbody(jg, accs_in):
        a = [av[pl.ds((jlo + jg * _JU + u) * _L, _L)] for u in range(_JU)]

        def kbody(kg, accs2):
            accs2 = list(accs2)
            for v in range(_KU):
                nb = nbv[pl.ds((kg * _KU + v) * _L, _L)]
                for u in range(_JU):
                    accs2[u] = accs2[u] + jnp.maximum(nb - a[u], 0.0)
            return tuple(accs2)

        return lax.fori_loop(0, _COLS // _KU, kbody, accs_in)

    accs = lax.fori_loop(0, _JHALF // _JU, jbody, (zeros,) * _JU)

    loss_vec = accs[0]
    for u in range(1, _JU):
        loss_vec = loss_vec + accs[u]

    lossv[...] = loss_vec
    pairsv[...] = pos_cnt * neg_cnt
    pltpu.sync_copy(lossv, loss_hbm.at[pl.ds(wid * _L, _L)])
    pltpu.sync_copy(pairsv, pairs_hbm.at[pl.ds(wid * _L, _L)])


_sc_call = pl.kernel(
    _sc_pair_loss,
    out_type=(
        jax.ShapeDtypeStruct((_NW * _L,), jnp.float32),
        jax.ShapeDtypeStruct((_NW * _L,), jnp.float32),
    ),
    mesh=plsc.VectorSubcoreMesh(core_axis_name="c", subcore_axis_name="s"),
    scratch_types=[
        pltpu.VMEM((_WVALS,), jnp.float32),
        pltpu.VMEM((_WVALS,), jnp.int32),
        pltpu.VMEM((_WVALS,), jnp.float32),
        pltpu.VMEM((_WVALS,), jnp.float32),
        pltpu.VMEM((_L,), jnp.float32),
        pltpu.VMEM((_L,), jnp.float32),
    ],
    compiler_params=pltpu.CompilerParams(needs_layout_passes=False),
)


def _tc_pair_kernel(yp_ref, yt_ref, loss_ref, pairs_ref):
    y = yp_ref[...]
    t = yt_ref[...]
    is_pos = t == 1
    is_neg = t == 0
    a = jnp.where(is_pos, y, jnp.inf)
    nb = jnp.where(is_neg, y + 1.0, -jnp.inf)

    _NACC = 4
    accs = [jnp.zeros((_TC_BLK, _COLS), jnp.float32) for _ in range(_NACC)]
    for j in range(_COLS):
        accs[j % _NACC] = accs[j % _NACC] + jnp.maximum(nb - a[:, j:j + 1],
                                                        0.0)
    tot = accs[0]
    for c in range(1, _NACC):
        tot = tot + accs[c]
    blk_loss = jnp.sum(tot)

    pos_cnt = jnp.sum(is_pos.astype(jnp.float32), axis=1)
    neg_cnt = jnp.sum(is_neg.astype(jnp.float32), axis=1)
    blk_pairs = jnp.sum(pos_cnt * neg_cnt)

    loss_ref[0, 0, 0] = blk_loss
    pairs_ref[0, 0, 0] = blk_pairs


_tc_blocks = (_ROWS - _SC_ROWS) // _TC_BLK
_tc_call = pl.pallas_call(
    _tc_pair_kernel,
    grid=(_tc_blocks,),
    in_specs=[
        pl.BlockSpec((_TC_BLK, _COLS), lambda b: (_SC_ROWS // _TC_BLK + b, 0)),
        pl.BlockSpec((_TC_BLK, _COLS), lambda b: (_SC_ROWS // _TC_BLK + b, 0)),
    ],
    out_specs=[
        pl.BlockSpec((1, 1, 1), lambda b: (b, 0, 0), memory_space=pltpu.SMEM),
        pl.BlockSpec((1, 1, 1), lambda b: (b, 0, 0), memory_space=pltpu.SMEM),
    ],
    out_shape=[
        jax.ShapeDtypeStruct((_tc_blocks, 1, 1), jnp.float32),
        jax.ShapeDtypeStruct((_tc_blocks, 1, 1), jnp.float32),
    ],
    compiler_params=pltpu.CompilerParams(
        dimension_semantics=("parallel",),
    ),
)


def _finalize(l_ref, p_ref, tl_ref, tp_ref, o_ref):
    s = jnp.sum(l_ref[...])
    t = jnp.sum(p_ref[...])
    for i in range(_tc_blocks):
        s = s + tl_ref[i, 0, 0]
        t = t + tp_ref[i, 0, 0]
    o_ref[0, 0] = jnp.where(t > 0.0, s / t, s)


_fin_call = pl.pallas_call(
    _finalize,
    in_specs=[
        pl.BlockSpec(memory_space=pltpu.VMEM),
        pl.BlockSpec(memory_space=pltpu.VMEM),
        pl.BlockSpec(memory_space=pltpu.SMEM),
        pl.BlockSpec(memory_space=pltpu.SMEM),
    ],
    out_shape=jax.ShapeDtypeStruct((1, 1), jnp.float32),
    out_specs=pl.BlockSpec(memory_space=pltpu.SMEM),
)


@jax.jit
def kernel(y_pred, y_true):
    yt32 = y_true.astype(jnp.int32)
    sc_loss, sc_pairs = _sc_call(y_pred.reshape(-1), yt32.reshape(-1))
    tc_loss, tc_pairs = _tc_call(y_pred, yt32)
    out = _fin_call(sc_loss.reshape(4, _COLS), sc_pairs.reshape(4, _COLS),
                    tc_loss, tc_pairs)
    return out[0, 0]


# speedup vs baseline: 1.0731x; 1.0731x over previous
import jax
import jax.numpy as jnp
from jax import lax
from jax.experimental import pallas as pl
from jax.experimental.pallas import tpu as pltpu
from jax.experimental.pallas import tpu_sc as plsc

_ROWS, _COLS = 1024, 128
_L = 16
_NC, _NS = 2, 16
_NW = _NC * _NS
_RPW = 16
_SC_ROWS = _NW * _RPW
_WVALS = _RPW * _COLS
_JU, _KU = 8, 4
_TC_BLK = 64


def _sc_pair_loss(yp_hbm, yt_hbm, loss_hbm, pairs_hbm,
                  ypv, ytv, av, nbv, lossv, pairsv):
    wid = lax.axis_index("s") * _NC + lax.axis_index("c")
    pltpu.sync_copy(yp_hbm.at[pl.ds(wid * _WVALS, _WVALS)], ypv)
    pltpu.sync_copy(yt_hbm.at[pl.ds(wid * _WVALS, _WVALS)], ytv)

    zeros = jnp.zeros((_L,), jnp.float32)
    row_off = lax.iota(jnp.int32, _L) * _COLS

    def prep(j, carry):
        pc, nc = carry
        idx = row_off + j
        y = plsc.load_gather(ypv, [idx])
        t = plsc.load_gather(ytv, [idx])
        is_pos = t == 1
        is_neg = t == 0
        av[pl.ds(j * _L, _L)] = jnp.where(is_pos, y, jnp.inf)
        nbv[pl.ds(j * _L, _L)] = jnp.where(is_neg, y + 1.0, -jnp.inf)
        pc = pc + jnp.where(is_pos, 1.0, 0.0)
        nc = nc + jnp.where(is_neg, 1.0, 0.0)
        return pc, nc

    pos_cnt, neg_cnt = lax.fori_loop(0, _COLS, prep, (zeros, zeros))

    def jbody(jg, accs_in):
        a = [av[pl.ds((jg * _JU + u) * _L, _L)] for u in range(_JU)]

        def kbody(kg, accs2):
            accs2 = list(accs2)
            for v in range(_KU):
                nb = nbv[pl.ds((kg * _KU + v) * _L, _L)]
                for u in range(_JU):
                    accs2[u] = accs2[u] + jnp.maximum(nb - a[u], 0.0)
            return tuple(accs2)

        return lax.fori_loop(0, _COLS // _KU, kbody, accs_in)

    accs = lax.fori_loop(0, _COLS // _JU, jbody, (zeros,) * _JU)

    loss_vec = accs[0]
    for u in range(1, _JU):
        loss_vec = loss_vec + accs[u]

    lossv[...] = loss_vec
    pairsv[...] = pos_cnt * neg_cnt
    pltpu.sync_copy(lossv, loss_hbm.at[pl.ds(wid * _L, _L)])
    pltpu.sync_copy(pairsv, pairs_hbm.at[pl.ds(wid * _L, _L)])


_sc_call = pl.kernel(
    _sc_pair_loss,
    out_type=(
        jax.ShapeDtypeStruct((_NW * _L,), jnp.float32),
        jax.ShapeDtypeStruct((_NW * _L,), jnp.float32),
    ),
    mesh=plsc.VectorSubcoreMesh(core_axis_name="c", subcore_axis_name="s"),
    scratch_types=[
        pltpu.VMEM((_WVALS,), jnp.float32),
        pltpu.VMEM((_WVALS,), jnp.int32),
        pltpu.VMEM((_WVALS,), jnp.float32),
        pltpu.VMEM((_WVALS,), jnp.float32),
        pltpu.VMEM((_L,), jnp.float32),
        pltpu.VMEM((_L,), jnp.float32),
    ],
    compiler_params=pltpu.CompilerParams(needs_layout_passes=False),
)


def _tc_pair_kernel(yp_ref, yt_ref, loss_ref, pairs_ref):
    y = yp_ref[...]
    t = yt_ref[...]
    is_pos = t == 1
    is_neg = t == 0
    a = jnp.where(is_pos, y, jnp.inf)
    nb = jnp.where(is_neg, y + 1.0, -jnp.inf)

    _NACC = 4
    accs = [jnp.zeros((_TC_BLK, _COLS), jnp.float32) for _ in range(_NACC)]
    for j in range(_COLS):
        accs[j % _NACC] = accs[j % _NACC] + jnp.maximum(nb - a[:, j:j + 1],
                                                        0.0)
    tot = accs[0]
    for c in range(1, _NACC):
        tot = tot + accs[c]
    blk_loss = jnp.sum(tot)

    pos_cnt = jnp.sum(is_pos.astype(jnp.float32), axis=1)
    neg_cnt = jnp.sum(is_neg.astype(jnp.float32), axis=1)
    blk_pairs = jnp.sum(pos_cnt * neg_cnt)

    loss_ref[0, 0, 0] = blk_loss
    pairs_ref[0, 0, 0] = blk_pairs


_tc_blocks = (_ROWS - _SC_ROWS) // _TC_BLK
_tc_call = pl.pallas_call(
    _tc_pair_kernel,
    grid=(_tc_blocks,),
    in_specs=[
        pl.BlockSpec((_TC_BLK, _COLS), lambda b: (_SC_ROWS // _TC_BLK + b, 0)),
        pl.BlockSpec((_TC_BLK, _COLS), lambda b: (_SC_ROWS // _TC_BLK + b, 0)),
    ],
    out_specs=[
        pl.BlockSpec((1, 1, 1), lambda b: (b, 0, 0), memory_space=pltpu.SMEM),
        pl.BlockSpec((1, 1, 1), lambda b: (b, 0, 0), memory_space=pltpu.SMEM),
    ],
    out_shape=[
        jax.ShapeDtypeStruct((_tc_blocks, 1, 1), jnp.float32),
        jax.ShapeDtypeStruct((_tc_blocks, 1, 1), jnp.float32),
    ],
    compiler_params=pltpu.CompilerParams(
        dimension_semantics=("parallel",),
    ),
)


def _finalize(l_ref, p_ref, tl_ref, tp_ref, o_ref):
    s = jnp.sum(l_ref[...])
    t = jnp.sum(p_ref[...])
    for i in range(_tc_blocks):
        s = s + tl_ref[i, 0, 0]
        t = t + tp_ref[i, 0, 0]
    o_ref[0, 0] = jnp.where(t > 0.0, s / t, s)


_fin_call = pl.pallas_call(
    _finalize,
    in_specs=[
        pl.BlockSpec(memory_space=pltpu.VMEM),
        pl.BlockSpec(memory_space=pltpu.VMEM),
        pl.BlockSpec(memory_space=pltpu.SMEM),
        pl.BlockSpec(memory_space=pltpu.SMEM),
    ],
    out_shape=jax.ShapeDtypeStruct((1, 1), jnp.float32),
    out_specs=pl.BlockSpec(memory_space=pltpu.SMEM),
)


@jax.jit
def kernel(y_pred, y_true):
    yt32 = y_true.astype(jnp.int32)
    sc_loss, sc_pairs = _sc_call(y_pred.reshape(-1), yt32.reshape(-1))
    tc_loss, tc_pairs = _tc_call(y_pred, yt32)
    out = _fin_call(sc_loss.reshape(4, _COLS), sc_pairs.reshape(4, _COLS),
                    tc_loss, tc_pairs)
    return out[0, 0]
